# Initial kernel scaffold; baseline (speedup 1.0000x reference)
import jax
import jax.numpy as jnp
from jax.experimental import pallas as pl


def kernel(edge_index, capacity, efeatures, W1, We1, al1, ar1, ae1, b1, W2, We2, al2, ar2, ae2, b2):
    raise NotImplementedError("write your pallas kernel here")


# speedup vs baseline: 115.5579x; 115.5579x over previous
import functools

import jax
import jax.numpy as jnp
from jax import lax
from jax.experimental import pallas as pl
from jax.experimental.pallas import tpu as pltpu
from jax.experimental.pallas import tpu_sc as plsc

N_NODES = 100000
N_EDGES = 1600000
E_TOTAL = N_EDGES + N_NODES
NP = 102400
NC, NS, L = 2, 16, 16
SL = NP // NS
CN = NP // (NC * NS)
EPT = N_EDGES // NS
EPW = N_EDGES // (NC * NS)

W0 = 2000
WM = 800
WS = 4000
WE = 2000

_MESH = plsc.VectorSubcoreMesh(
    core_axis_name="c", subcore_axis_name="s", num_cores=NC, num_subcores=NS)
_CPARAMS = pltpu.CompilerParams(needs_layout_passes=False)

_f32 = jnp.float32
_i32 = jnp.int32


def _leaky(x):
    return jnp.where(x >= 0, x, 0.2 * x)


def _fill(ref, n, value):
    def body(k, _):
        ref[pl.ds(k * L, L)] = jnp.full((L,), value, _f32)
        return 0
    lax.fori_loop(0, n // L, body, 0)


def _row16(cref, c):
    sel = jnp.full((L,), c, _i32) == 0
    return jnp.where(sel, cref[0, :], cref[1, :])


def _lgather(x, idx):
    dnums = lax.GatherDimensionNumbers(
        offset_dims=(), collapsed_slice_dims=(0,), start_index_map=(0,))
    return lax.gather(x, idx[:, None], dnums, slice_sizes=(1,),
                      mode=lax.GatherScatterMode.PROMISE_IN_BOUNDS)


def _sqrt16(v):
    i = plsc.bitcast(v, _i32)
    y = plsc.bitcast((i >> 1) + 0x1FBD1DF5, _f32)
    for _ in range(3):
        y = 0.5 * (y + v / y)
    return y


@functools.partial(
    pl.kernel,
    out_type=jax.ShapeDtypeStruct((NP,), _f32),
    mesh=_MESH,
    compiler_params=_CPARAMS,
    scratch_types=[
        pltpu.VMEM_SHARED((NP,), _f32),
        pltpu.VMEM((W0,), _i32),
        pltpu.VMEM((W0,), _f32),
        pltpu.VMEM((SL,), _f32),
    ],
)
def _k_sumcap(dst_h, cap_h, sc_out, spm_sc, wdst, wcap, zb):
    c = lax.axis_index("c")
    s = lax.axis_index("s")
    base_n = s * SL
    _fill(zb, SL, 0.0)
    pltpu.sync_copy(zb, spm_sc.at[pl.ds(base_n, SL)])
    plsc.subcore_barrier()

    def win(w, _):
        off = s * EPT + w * W0
        pltpu.sync_copy(dst_h.at[pl.ds(off, W0)], wdst)
        pltpu.sync_copy(cap_h.at[pl.ds(off, W0)], wcap)
        pltpu.sync_copy(wcap, spm_sc.at[wdst], add=True)
        return 0
    lax.fori_loop(0, EPT // W0, win, 0)
    plsc.subcore_barrier()

    @pl.when(c == 0)
    def _():
        pltpu.sync_copy(spm_sc.at[pl.ds(base_n, SL)], zb)
        pltpu.sync_copy(zb, sc_out.at[pl.ds(base_n, SL)])


@functools.partial(
    pl.kernel,
    out_type=jax.ShapeDtypeStruct((NC, NP), _f32),
    mesh=_MESH,
    compiler_params=_CPARAMS,
    scratch_types=[
        pltpu.VMEM_SHARED((NP,), _f32),
        pltpu.HBM((NC, NS, NP), _f32),
        pltpu.VMEM((NP,), _f32),
        pltpu.VMEM((WM,), _i32),
        pltpu.VMEM((WM,), _i32),
        pltpu.VMEM((WM,), _f32),
        pltpu.VMEM((WM,), _f32),
        pltpu.VMEM((WM,), _f32),
        pltpu.VMEM((SL,), _f32),
        pltpu.VMEM((SL,), _f32),
        pltpu.VMEM((2, 16), _f32),
        pltpu.VMEM((2, 16), _f32),
        pltpu.VMEM((2, 16), _f32),
    ],
)
def _k_segmax(src_h, dst_h, efe_h, efl_h, val2_h, cl_h, cr_h, ce_h, m_out,
              spm_v, spm_mg, priv, wsrc, wdst, wef, vs, vd, ab, mb,
              clb, crb, ceb):
    c = lax.axis_index("c")
    s = lax.axis_index("s")
    base_n = s * SL

    pltpu.sync_copy(val2_h.at[c, pl.ds(base_n, SL)], ab)
    pltpu.sync_copy(ab, spm_v.at[pl.ds(base_n, SL)])
    pltpu.sync_copy(cl_h, clb)
    pltpu.sync_copy(cr_h, crb)
    pltpu.sync_copy(ce_h, ceb)
    plsc.subcore_barrier()

    clv = _row16(clb, c)
    crv = _row16(crb, c)
    cev = _row16(ceb, c)

    _fill(priv, NP, -1e30)

    def win(w, _):
        off = s * EPT + w * WM
        pltpu.sync_copy(src_h.at[pl.ds(off, WM)], wsrc)
        pltpu.sync_copy(dst_h.at[pl.ds(off, WM)], wdst)
        pltpu.sync_copy(efe_h.at[pl.ds(off, WM)], wef)
        pltpu.sync_copy(spm_v.at[wsrc], vs)
        pltpu.sync_copy(spm_v.at[wdst], vd)

        def vec(k, _):
            sl = pl.ds(k * L, L)
            e = _leaky(clv * vs[sl] + crv * vd[sl] + cev * wef[sl])
            sk = wdst[sl]
            sv = e
            iota = lax.iota(_i32, L)
            seen = iota < 0
            for sft in range(1, L):
                pidx = (iota - sft) & (L - 1)
                rk = _lgather(sk, pidx)
                rv = _lgather(sv, pidx)
                eq = rk == sk
                sv = jnp.where(eq, jnp.maximum(sv, rv), sv)
                seen = seen | (eq & (iota >= sft))
            cur = plsc.load_gather(priv, [sk])
            plsc.store_scatter(priv, [sk], jnp.maximum(cur, sv),
                               mask=jnp.logical_not(seen))
            return 0
        lax.fori_loop(0, WM // L, vec, 0)
        return 0
    lax.fori_loop(0, EPT // WM, win, 0)

    pltpu.sync_copy(priv, spm_mg.at[c, s])
    plsc.subcore_barrier()

    _fill(ab, SL, -1e30)
    for tt in range(NS):
        pltpu.sync_copy(spm_mg.at[c, tt, pl.ds(base_n, SL)], mb)

        def fold(k, _):
            sl = pl.ds(k * L, L)
            ab[sl] = jnp.maximum(ab[sl], mb[sl])
            return 0
        lax.fori_loop(0, SL // L, fold, 0)

    pltpu.sync_copy(spm_v.at[pl.ds(base_n, SL)], mb)

    def chunk(cc, _):
        pltpu.sync_copy(efl_h.at[pl.ds(base_n + cc * WM, WM)], wef)

        def fold3(k, _):
            sl = pl.ds(k * L, L)
            gsl = pl.ds(cc * WM + k * L, L)
            el = _leaky((clv + crv) * mb[gsl] + cev * wef[sl])
            ab[gsl] = jnp.maximum(ab[gsl], el)
            return 0
        lax.fori_loop(0, WM // L, fold3, 0)
        return 0
    lax.fori_loop(0, SL // WM, chunk, 0)

    pltpu.sync_copy(ab, m_out.at[c, pl.ds(base_n, SL)])


@functools.partial(
    pl.kernel,
    out_type=jax.ShapeDtypeStruct((NC, NP), _f32),
    mesh=_MESH,
    compiler_params=_CPARAMS,
    scratch_types=[
        pltpu.VMEM_SHARED((NP,), _f32),
        pltpu.VMEM_SHARED((NP,), _f32),
        pltpu.VMEM_SHARED((NP,), _f32),
        pltpu.VMEM_SHARED((NP,), _f32),
        pltpu.VMEM((WS,), _i32),
        pltpu.VMEM((WS,), _i32),
        pltpu.VMEM((WS,), _f32),
        pltpu.VMEM((WS,), _f32),
        pltpu.VMEM((WS,), _f32),
        pltpu.VMEM((WS,), _f32),
        pltpu.VMEM((WS,), _f32),
        pltpu.VMEM((WS,), _f32),
        pltpu.VMEM((SL,), _f32),
        pltpu.VMEM((SL,), _f32),
        pltpu.VMEM((SL,), _f32),
        pltpu.VMEM((SL,), _f32),
        pltpu.VMEM((SL,), _f32),
        pltpu.VMEM((2, 16), _f32),
        pltpu.VMEM((2, 16), _f32),
        pltpu.VMEM((2, 16), _f32),
    ],
)
def _k_segsum(src_h, dst_h, efe_h, efl_h, val2_h, m2_h, cl_h, cr_h, ce_h,
              s_out, spm_v, spm_m, spm_d, spm_u, wsrc, wdst, wef, vs, vd, vm,
              exb, uxb, b1, b2, b3, b4, b5, clb, crb, ceb):
    c = lax.axis_index("c")
    s = lax.axis_index("s")
    base_n = s * SL

    pltpu.sync_copy(val2_h.at[c, pl.ds(base_n, SL)], b1)
    pltpu.sync_copy(b1, spm_v.at[pl.ds(base_n, SL)])
    pltpu.sync_copy(m2_h.at[c, pl.ds(base_n, SL)], b2)
    pltpu.sync_copy(b2, spm_m.at[pl.ds(base_n, SL)])
    _fill(b3, SL, 0.0)
    pltpu.sync_copy(b3, spm_d.at[pl.ds(base_n, SL)])
    pltpu.sync_copy(b3, spm_u.at[pl.ds(base_n, SL)])
    pltpu.sync_copy(cl_h, clb)
    pltpu.sync_copy(cr_h, crb)
    pltpu.sync_copy(ce_h, ceb)
    plsc.subcore_barrier()

    clv = _row16(clb, c)
    crv = _row16(crb, c)
    cev = _row16(ceb, c)

    def win(w, _):
        off = s * EPT + w * WS
        pltpu.sync_copy(src_h.at[pl.ds(off, WS)], wsrc)
        pltpu.sync_copy(dst_h.at[pl.ds(off, WS)], wdst)
        pltpu.sync_copy(efe_h.at[pl.ds(off, WS)], wef)
        pltpu.sync_copy(spm_v.at[wsrc], vs)
        pltpu.sync_copy(spm_v.at[wdst], vd)
        pltpu.sync_copy(spm_m.at[wdst], vm)

        def vec(k, _):
            sl = pl.ds(k * L, L)
            vsv = vs[sl]
            e = _leaky(clv * vsv + crv * vd[sl] + cev * wef[sl])
            ex = jnp.exp(e - vm[sl])
            exb[sl] = ex
            uxb[sl] = ex * vsv
            return 0
        lax.fori_loop(0, WS // L, vec, 0)
        pltpu.sync_copy(exb, spm_d.at[wdst], add=True)
        pltpu.sync_copy(uxb, spm_u.at[wdst], add=True)
        return 0
    lax.fori_loop(0, EPT // WS, win, 0)
    plsc.subcore_barrier()

    pltpu.sync_copy(spm_d.at[pl.ds(base_n, SL)], b1)
    pltpu.sync_copy(spm_u.at[pl.ds(base_n, SL)], b2)
    pltpu.sync_copy(spm_v.at[pl.ds(base_n, SL)], b3)
    pltpu.sync_copy(spm_m.at[pl.ds(base_n, SL)], b4)
    pltpu.sync_copy(efl_h.at[pl.ds(base_n, SL)], b5)

    def rd(k, _):
        sl = pl.ds(k * L, L)
        v = b3[sl]
        el = _leaky((clv + crv) * v + cev * b5[sl])
        exl = jnp.exp(el - b4[sl])
        b1[sl] = (b2[sl] + exl * v) / (b1[sl] + exl)
        return 0
    lax.fori_loop(0, SL // L, rd, 0)
    pltpu.sync_copy(b1, s_out.at[c, pl.ds(base_n, SL)])


@functools.partial(
    pl.kernel,
    out_type=jax.ShapeDtypeStruct((NC, NP), _f32),
    mesh=_MESH,
    compiler_params=_CPARAMS,
    scratch_types=[
        pltpu.VMEM((CN,), _f32),
        pltpu.VMEM((CN,), _f32),
        pltpu.VMEM((CN,), _f32),
        pltpu.VMEM((CN,), _f32),
        pltpu.VMEM((16, 16), _f32),
        pltpu.VMEM((16, 16), _f32),
        pltpu.VMEM((16, 16), _f32),
        pltpu.VMEM((16, 16), _f32),
        pltpu.VMEM((16, 16), _f32),
    ],
)
def _k_node(s_h, w0_h, w1_h, bs_h, w20_h, w21_h, g_out,
            s0b, s1b, g0b, g1b, cw0, cw1, cbs, cw20, cw21):
    c = lax.axis_index("c")
    s = lax.axis_index("s")
    wid = s * NC + c
    base = wid * CN
    pltpu.sync_copy(s_h.at[0, pl.ds(base, CN)], s0b)
    pltpu.sync_copy(s_h.at[1, pl.ds(base, CN)], s1b)
    pltpu.sync_copy(w0_h, cw0)
    pltpu.sync_copy(w1_h, cw1)
    pltpu.sync_copy(bs_h, cbs)
    pltpu.sync_copy(w20_h, cw20)
    pltpu.sync_copy(w21_h, cw21)

    def vec(k, _):
        sl = pl.ds(k * L, L)
        s0 = s0b[sl]
        s1 = s1b[sl]
        g0 = jnp.zeros((L,), _f32)
        g1 = jnp.zeros((L,), _f32)
        for f in range(16):
            t = jnp.maximum(cw0[f, :] * s0 + cw1[f, :] * s1 + cbs[f, :], 0.0)
            g0 = g0 + t * cw20[f, :]
            g1 = g1 + t * cw21[f, :]
        g0b[sl] = g0
        g1b[sl] = g1
        return 0
    lax.fori_loop(0, CN // L, vec, 0)
    pltpu.sync_copy(g0b, g_out.at[0, pl.ds(base, CN)])
    pltpu.sync_copy(g1b, g_out.at[1, pl.ds(base, CN)])


@functools.partial(
    pl.kernel,
    out_type=jax.ShapeDtypeStruct((E_TOTAL,), _f32),
    mesh=_MESH,
    compiler_params=_CPARAMS,
    scratch_types=[
        pltpu.VMEM_SHARED((NP,), _f32),
        pltpu.VMEM((SL,), _f32),
        pltpu.VMEM((SL,), _f32),
        pltpu.VMEM((CN,), _f32),
        pltpu.VMEM((WE,), _i32),
        pltpu.VMEM((WE,), _i32),
        pltpu.VMEM((WE,), _f32),
        pltpu.VMEM((WE,), _f32),
        pltpu.VMEM((WE,), _f32),
        pltpu.VMEM((16,), _f32),
    ],
)
def _k_out(src_h, dst_h, t_h, b2s_h, out_h, spm_x, t0b, t1b, ob,
           wsrc, wdst, xs, xd, oe, b2b):
    c = lax.axis_index("c")
    s = lax.axis_index("s")
    wid = s * NC + c
    base_n = s * SL
    pltpu.sync_copy(t_h.at[0, pl.ds(base_n, SL)], t0b)
    pltpu.sync_copy(t_h.at[1, pl.ds(base_n, SL)], t1b)
    pltpu.sync_copy(b2s_h, b2b)
    b2v = b2b[...]

    def vec(k, _):
        sl = pl.ds(k * L, L)
        t0b[sl] = t0b[sl] + t1b[sl] + b2v
        return 0
    lax.fori_loop(0, SL // L, vec, 0)
    pltpu.sync_copy(t0b, spm_x.at[pl.ds(base_n, SL)])

    def vecs(k, _):
        sl = pl.ds(k * L, L)
        x = t0b[pl.ds(c * CN + k * L, L)]
        ob[sl] = _sqrt16(x * x)
        return 0
    lax.fori_loop(0, CN // L, vecs, 0)
    start = wid * CN

    @pl.when(wid < (NC * NS - 1))
    def _():
        pltpu.sync_copy(ob, out_h.at[pl.ds(N_EDGES + start, CN)])

    @pl.when(wid == (NC * NS - 1))
    def _():
        tail = N_NODES - (NC * NS - 1) * CN
        pltpu.sync_copy(ob.at[pl.ds(0, tail)],
                        out_h.at[pl.ds(N_EDGES + start, tail)])

    plsc.subcore_barrier()

    def win(w, _):
        off = wid * EPW + w * WE
        pltpu.sync_copy(src_h.at[pl.ds(off, WE)], wsrc)
        pltpu.sync_copy(dst_h.at[pl.ds(off, WE)], wdst)
        pltpu.sync_copy(spm_x.at[wsrc], xs)
        pltpu.sync_copy(spm_x.at[wdst], xd)

        def vec2(k, _):
            sl = pl.ds(k * L, L)
            oe[sl] = _sqrt16(xs[sl] * xd[sl])
            return 0
        lax.fori_loop(0, WE // L, vec2, 0)
        pltpu.sync_copy(oe, out_h.at[pl.ds(off, WE)])
        return 0
    lax.fori_loop(0, EPW // WE, win, 0)


def kernel(edge_index, capacity, efeatures, W1, We1, al1, ar1, ae1, b1,
           W2, We2, al2, ar2, ae2, b2):
    src = edge_index[0].astype(_i32)
    dst = edge_index[1].astype(_i32)
    cap = capacity[:, 0]
    efe = efeatures[:N_EDGES, 0]
    efl = jnp.pad(efeatures[N_EDGES:, 0], (0, NP - N_NODES))

    W1h = W1.reshape(2, 16)
    cl1 = jnp.sum(W1h * al1, axis=1)
    cr1 = jnp.sum(W1h * ar1, axis=1)
    ce1 = jnp.sum(We1.reshape(2, 16) * ae1, axis=1)
    bsum = b1[:16] + b1[16:]
    a2l = al2[:, 0]
    a2r = ar2[:, 0]
    ce2 = We2[0] * ae2[:, 0]
    b2s = b2[0] + b2[1]

    def spl2(v):
        return jnp.broadcast_to(v[:, None], (2, 16)).astype(_f32)

    def spl16(v):
        return jnp.broadcast_to(v[:, None], (16, 16)).astype(_f32)

    sc = _k_sumcap(dst, cap)
    v1 = jnp.broadcast_to(sc[None, :], (NC, NP))
    m1 = _k_segmax(src, dst, efe, efl, v1, spl2(cl1), spl2(cr1), spl2(ce1))
    s1 = _k_segsum(src, dst, efe, efl, v1, m1,
                   spl2(cl1), spl2(cr1), spl2(ce1))
    g = _k_node(s1, spl16(W1h[0]), spl16(W1h[1]), spl16(bsum),
                spl16(W2[:, 0]), spl16(W2[:, 1]))
    m2 = _k_segmax(src, dst, efe, efl, g, spl2(a2l), spl2(a2r), spl2(ce2))
    t2 = _k_segsum(src, dst, efe, efl, g, m2, spl2(a2l), spl2(a2r), spl2(ce2))
    out = _k_out(src, dst, t2, jnp.broadcast_to(b2s, (16,)).astype(_f32))
    return out[:, None]


# speedup vs baseline: 121.7823x; 1.0539x over previous
import functools

import jax
import jax.numpy as jnp
from jax import lax
from jax.experimental import pallas as pl
from jax.experimental.pallas import tpu as pltpu
from jax.experimental.pallas import tpu_sc as plsc

N_NODES = 100000
N_EDGES = 1600000
E_TOTAL = N_EDGES + N_NODES
NP = 102400
NC, NS, L = 2, 16, 16
SL = NP // NS
CN = NP // (NC * NS)
EPT = N_EDGES // NS
EPW = N_EDGES // (NC * NS)

W0 = 2000
WM = 800
WS = 4000
WE = 2000

_MESH = plsc.VectorSubcoreMesh(
    core_axis_name="c", subcore_axis_name="s", num_cores=NC, num_subcores=NS)
_CPARAMS = pltpu.CompilerParams(needs_layout_passes=False)

_f32 = jnp.float32
_i32 = jnp.int32


def _leaky(x):
    return jnp.where(x >= 0, x, 0.2 * x)


def _fill(ref, n, value):
    def body(k, _):
        ref[pl.ds(k * L, L)] = jnp.full((L,), value, _f32)
        return 0
    lax.fori_loop(0, n // L, body, 0)


def _row16(cref, c):
    sel = jnp.full((L,), c, _i32) == 0
    return jnp.where(sel, cref[0, :], cref[1, :])


def _lgather(x, idx):
    dnums = lax.GatherDimensionNumbers(
        offset_dims=(), collapsed_slice_dims=(0,), start_index_map=(0,))
    return lax.gather(x, idx[:, None], dnums, slice_sizes=(1,),
                      mode=lax.GatherScatterMode.PROMISE_IN_BOUNDS)


def _sqrt16(v):
    i = plsc.bitcast(v, _i32)
    y = plsc.bitcast((i >> 1) + 0x1FBD1DF5, _f32)
    for _ in range(3):
        y = 0.5 * (y + v / y)
    return y


@functools.partial(
    pl.kernel,
    out_type=jax.ShapeDtypeStruct((NP,), _f32),
    mesh=_MESH,
    compiler_params=_CPARAMS,
    scratch_types=[
        pltpu.VMEM_SHARED((NP,), _f32),
        pltpu.VMEM((W0,), _i32),
        pltpu.VMEM((W0,), _f32),
        pltpu.VMEM((SL,), _f32),
    ],
)
def _k_sumcap(dst_h, cap_h, sc_out, spm_sc, wdst, wcap, zb):
    c = lax.axis_index("c")
    s = lax.axis_index("s")
    base_n = s * SL
    _fill(zb, SL, 0.0)
    pltpu.sync_copy(zb, spm_sc.at[pl.ds(base_n, SL)])
    plsc.subcore_barrier()

    def win(w, _):
        off = s * EPT + w * W0
        pltpu.sync_copy(dst_h.at[pl.ds(off, W0)], wdst)
        pltpu.sync_copy(cap_h.at[pl.ds(off, W0)], wcap)
        pltpu.sync_copy(wcap, spm_sc.at[wdst], add=True)
        return 0
    lax.fori_loop(0, EPT // W0, win, 0)
    plsc.subcore_barrier()

    @pl.when(c == 0)
    def _():
        pltpu.sync_copy(spm_sc.at[pl.ds(base_n, SL)], zb)
        pltpu.sync_copy(zb, sc_out.at[pl.ds(base_n, SL)])


@functools.partial(
    pl.kernel,
    out_type=jax.ShapeDtypeStruct((NC, NP), _f32),
    mesh=_MESH,
    compiler_params=_CPARAMS,
    scratch_types=[
        pltpu.VMEM_SHARED((NP,), _f32),
        pltpu.HBM((NC, NS, NP), _f32),
        pltpu.VMEM((NP,), _f32),
        pltpu.VMEM((WM,), _i32),
        pltpu.VMEM((WM,), _i32),
        pltpu.VMEM((WM,), _f32),
        pltpu.VMEM((WM,), _f32),
        pltpu.VMEM((WM,), _f32),
        pltpu.VMEM((SL,), _f32),
        pltpu.VMEM((SL,), _f32),
        pltpu.VMEM((2, 16), _f32),
        pltpu.VMEM((2, 16), _f32),
        pltpu.VMEM((2, 16), _f32),
    ],
)
def _k_segmax(src_h, dst_h, efe_h, efl_h, val2_h, cl_h, cr_h, ce_h, m_out,
              spm_v, spm_mg, priv, wsrc, wdst, wef, vs, vd, ab, mb,
              clb, crb, ceb):
    c = lax.axis_index("c")
    s = lax.axis_index("s")
    base_n = s * SL

    pltpu.sync_copy(val2_h.at[c, pl.ds(base_n, SL)], ab)
    pltpu.sync_copy(ab, spm_v.at[pl.ds(base_n, SL)])
    pltpu.sync_copy(cl_h, clb)
    pltpu.sync_copy(cr_h, crb)
    pltpu.sync_copy(ce_h, ceb)
    plsc.subcore_barrier()

    clv = _row16(clb, c)
    crv = _row16(crb, c)
    cev = _row16(ceb, c)

    _fill(priv, NP, -1e30)

    def win(w, _):
        off = s * EPT + w * WM
        pltpu.sync_copy(src_h.at[pl.ds(off, WM)], wsrc)
        pltpu.sync_copy(dst_h.at[pl.ds(off, WM)], wdst)
        pltpu.sync_copy(efe_h.at[pl.ds(off, WM)], wef)
        pltpu.sync_copy(spm_v.at[wsrc], vs)
        pltpu.sync_copy(spm_v.at[wdst], vd)

        def vec(k, _):
            sl = pl.ds(k * L, L)
            e = _leaky(clv * vs[sl] + crv * vd[sl] + cev * wef[sl])
            sk = wdst[sl]
            cur = plsc.load_gather(priv, [sk])
            plsc.store_scatter(priv, [sk], jnp.maximum(cur, e))
            after = plsc.load_gather(priv, [sk])

            @pl.when(jnp.any(e > after))
            def _():
                sv = e
                iota = lax.iota(_i32, L)
                seen = iota < 0
                for sft in range(1, L):
                    pidx = (iota - sft) & (L - 1)
                    rk = _lgather(sk, pidx)
                    rv = _lgather(sv, pidx)
                    eq = rk == sk
                    sv = jnp.where(eq, jnp.maximum(sv, rv), sv)
                    seen = seen | (eq & (iota >= sft))
                cur2 = plsc.load_gather(priv, [sk])
                plsc.store_scatter(priv, [sk], jnp.maximum(cur2, sv),
                                   mask=jnp.logical_not(seen))
            return 0
        lax.fori_loop(0, WM // L, vec, 0)
        return 0
    lax.fori_loop(0, EPT // WM, win, 0)

    pltpu.sync_copy(priv, spm_mg.at[c, s])
    plsc.subcore_barrier()

    _fill(ab, SL, -1e30)
    for tt in range(NS):
        pltpu.sync_copy(spm_mg.at[c, tt, pl.ds(base_n, SL)], mb)

        def fold(k, _):
            sl = pl.ds(k * L, L)
            ab[sl] = jnp.maximum(ab[sl], mb[sl])
            return 0
        lax.fori_loop(0, SL // L, fold, 0)

    pltpu.sync_copy(spm_v.at[pl.ds(base_n, SL)], mb)

    def chunk(cc, _):
        pltpu.sync_copy(efl_h.at[pl.ds(base_n + cc * WM, WM)], wef)

        def fold3(k, _):
            sl = pl.ds(k * L, L)
            gsl = pl.ds(cc * WM + k * L, L)
            el = _leaky((clv + crv) * mb[gsl] + cev * wef[sl])
            ab[gsl] = jnp.maximum(ab[gsl], el)
            return 0
        lax.fori_loop(0, WM // L, fold3, 0)
        return 0
    lax.fori_loop(0, SL // WM, chunk, 0)

    pltpu.sync_copy(ab, m_out.at[c, pl.ds(base_n, SL)])


@functools.partial(
    pl.kernel,
    out_type=jax.ShapeDtypeStruct((NC, NP), _f32),
    mesh=_MESH,
    compiler_params=_CPARAMS,
    scratch_types=[
        pltpu.VMEM_SHARED((NP,), _f32),
        pltpu.VMEM_SHARED((NP,), _f32),
        pltpu.VMEM_SHARED((NP,), _f32),
        pltpu.VMEM_SHARED((NP,), _f32),
        pltpu.VMEM((WS,), _i32),
        pltpu.VMEM((WS,), _i32),
        pltpu.VMEM((WS,), _f32),
        pltpu.VMEM((WS,), _f32),
        pltpu.VMEM((WS,), _f32),
        pltpu.VMEM((WS,), _f32),
        pltpu.VMEM((WS,), _f32),
        pltpu.VMEM((WS,), _f32),
        pltpu.VMEM((SL,), _f32),
        pltpu.VMEM((SL,), _f32),
        pltpu.VMEM((SL,), _f32),
        pltpu.VMEM((SL,), _f32),
        pltpu.VMEM((SL,), _f32),
        pltpu.VMEM((2, 16), _f32),
        pltpu.VMEM((2, 16), _f32),
        pltpu.VMEM((2, 16), _f32),
    ],
)
def _k_segsum(src_h, dst_h, efe_h, efl_h, val2_h, m2_h, cl_h, cr_h, ce_h,
              s_out, spm_v, spm_m, spm_d, spm_u, wsrc, wdst, wef, vs, vd, vm,
              exb, uxb, b1, b2, b3, b4, b5, clb, crb, ceb):
    c = lax.axis_index("c")
    s = lax.axis_index("s")
    base_n = s * SL

    pltpu.sync_copy(val2_h.at[c, pl.ds(base_n, SL)], b1)
    pltpu.sync_copy(b1, spm_v.at[pl.ds(base_n, SL)])
    pltpu.sync_copy(m2_h.at[c, pl.ds(base_n, SL)], b2)
    pltpu.sync_copy(b2, spm_m.at[pl.ds(base_n, SL)])
    _fill(b3, SL, 0.0)
    pltpu.sync_copy(b3, spm_d.at[pl.ds(base_n, SL)])
    pltpu.sync_copy(b3, spm_u.at[pl.ds(base_n, SL)])
    pltpu.sync_copy(cl_h, clb)
    pltpu.sync_copy(cr_h, crb)
    pltpu.sync_copy(ce_h, ceb)
    plsc.subcore_barrier()

    clv = _row16(clb, c)
    crv = _row16(crb, c)
    cev = _row16(ceb, c)

    def win(w, _):
        off = s * EPT + w * WS
        pltpu.sync_copy(src_h.at[pl.ds(off, WS)], wsrc)
        pltpu.sync_copy(dst_h.at[pl.ds(off, WS)], wdst)
        pltpu.sync_copy(efe_h.at[pl.ds(off, WS)], wef)
        pltpu.sync_copy(spm_v.at[wsrc], vs)
        pltpu.sync_copy(spm_v.at[wdst], vd)
        pltpu.sync_copy(spm_m.at[wdst], vm)

        def vec(k, _):
            sl = pl.ds(k * L, L)
            vsv = vs[sl]
            e = _leaky(clv * vsv + crv * vd[sl] + cev * wef[sl])
            ex = jnp.exp(e - vm[sl])
            exb[sl] = ex
            uxb[sl] = ex * vsv
            return 0
        lax.fori_loop(0, WS // L, vec, 0)
        pltpu.sync_copy(exb, spm_d.at[wdst], add=True)
        pltpu.sync_copy(uxb, spm_u.at[wdst], add=True)
        return 0
    lax.fori_loop(0, EPT // WS, win, 0)
    plsc.subcore_barrier()

    pltpu.sync_copy(spm_d.at[pl.ds(base_n, SL)], b1)
    pltpu.sync_copy(spm_u.at[pl.ds(base_n, SL)], b2)
    pltpu.sync_copy(spm_v.at[pl.ds(base_n, SL)], b3)
    pltpu.sync_copy(spm_m.at[pl.ds(base_n, SL)], b4)
    pltpu.sync_copy(efl_h.at[pl.ds(base_n, SL)], b5)

    def rd(k, _):
        sl = pl.ds(k * L, L)
        v = b3[sl]
        el = _leaky((clv + crv) * v + cev * b5[sl])
        exl = jnp.exp(el - b4[sl])
        b1[sl] = (b2[sl] + exl * v) / (b1[sl] + exl)
        return 0
    lax.fori_loop(0, SL // L, rd, 0)
    pltpu.sync_copy(b1, s_out.at[c, pl.ds(base_n, SL)])


@functools.partial(
    pl.kernel,
    out_type=jax.ShapeDtypeStruct((NC, NP), _f32),
    mesh=_MESH,
    compiler_params=_CPARAMS,
    scratch_types=[
        pltpu.VMEM((CN,), _f32),
        pltpu.VMEM((CN,), _f32),
        pltpu.VMEM((CN,), _f32),
        pltpu.VMEM((CN,), _f32),
        pltpu.VMEM((16, 16), _f32),
        pltpu.VMEM((16, 16), _f32),
        pltpu.VMEM((16, 16), _f32),
        pltpu.VMEM((16, 16), _f32),
        pltpu.VMEM((16, 16), _f32),
    ],
)
def _k_node(s_h, w0_h, w1_h, bs_h, w20_h, w21_h, g_out,
            s0b, s1b, g0b, g1b, cw0, cw1, cbs, cw20, cw21):
    c = lax.axis_index("c")
    s = lax.axis_index("s")
    wid = s * NC + c
    base = wid * CN
    pltpu.sync_copy(s_h.at[0, pl.ds(base, CN)], s0b)
    pltpu.sync_copy(s_h.at[1, pl.ds(base, CN)], s1b)
    pltpu.sync_copy(w0_h, cw0)
    pltpu.sync_copy(w1_h, cw1)
    pltpu.sync_copy(bs_h, cbs)
    pltpu.sync_copy(w20_h, cw20)
    pltpu.sync_copy(w21_h, cw21)

    def vec(k, _):
        sl = pl.ds(k * L, L)
        s0 = s0b[sl]
        s1 = s1b[sl]
        g0 = jnp.zeros((L,), _f32)
        g1 = jnp.zeros((L,), _f32)
        for f in range(16):
            t = jnp.maximum(cw0[f, :] * s0 + cw1[f, :] * s1 + cbs[f, :], 0.0)
            g0 = g0 + t * cw20[f, :]
            g1 = g1 + t * cw21[f, :]
        g0b[sl] = g0
        g1b[sl] = g1
        return 0
    lax.fori_loop(0, CN // L, vec, 0)
    pltpu.sync_copy(g0b, g_out.at[0, pl.ds(base, CN)])
    pltpu.sync_copy(g1b, g_out.at[1, pl.ds(base, CN)])


@functools.partial(
    pl.kernel,
    out_type=jax.ShapeDtypeStruct((E_TOTAL,), _f32),
    mesh=_MESH,
    compiler_params=_CPARAMS,
    scratch_types=[
        pltpu.VMEM_SHARED((NP,), _f32),
        pltpu.VMEM((SL,), _f32),
        pltpu.VMEM((SL,), _f32),
        pltpu.VMEM((CN,), _f32),
        pltpu.VMEM((WE,), _i32),
        pltpu.VMEM((WE,), _i32),
        pltpu.VMEM((WE,), _f32),
        pltpu.VMEM((WE,), _f32),
        pltpu.VMEM((WE,), _f32),
        pltpu.VMEM((16,), _f32),
    ],
)
def _k_out(src_h, dst_h, t_h, b2s_h, out_h, spm_x, t0b, t1b, ob,
           wsrc, wdst, xs, xd, oe, b2b):
    c = lax.axis_index("c")
    s = lax.axis_index("s")
    wid = s * NC + c
    base_n = s * SL
    pltpu.sync_copy(t_h.at[0, pl.ds(base_n, SL)], t0b)
    pltpu.sync_copy(t_h.at[1, pl.ds(base_n, SL)], t1b)
    pltpu.sync_copy(b2s_h, b2b)
    b2v = b2b[...]

    def vec(k, _):
        sl = pl.ds(k * L, L)
        t0b[sl] = t0b[sl] + t1b[sl] + b2v
        return 0
    lax.fori_loop(0, SL // L, vec, 0)
    pltpu.sync_copy(t0b, spm_x.at[pl.ds(base_n, SL)])

    def vecs(k, _):
        sl = pl.ds(k * L, L)
        x = t0b[pl.ds(c * CN + k * L, L)]
        ob[sl] = _sqrt16(x * x)
        return 0
    lax.fori_loop(0, CN // L, vecs, 0)
    start = wid * CN

    @pl.when(wid < (NC * NS - 1))
    def _():
        pltpu.sync_copy(ob, out_h.at[pl.ds(N_EDGES + start, CN)])

    @pl.when(wid == (NC * NS - 1))
    def _():
        tail = N_NODES - (NC * NS - 1) * CN
        pltpu.sync_copy(ob.at[pl.ds(0, tail)],
                        out_h.at[pl.ds(N_EDGES + start, tail)])

    plsc.subcore_barrier()

    def win(w, _):
        off = wid * EPW + w * WE
        pltpu.sync_copy(src_h.at[pl.ds(off, WE)], wsrc)
        pltpu.sync_copy(dst_h.at[pl.ds(off, WE)], wdst)
        pltpu.sync_copy(spm_x.at[wsrc], xs)
        pltpu.sync_copy(spm_x.at[wdst], xd)

        def vec2(k, _):
            sl = pl.ds(k * L, L)
            oe[sl] = _sqrt16(xs[sl] * xd[sl])
            return 0
        lax.fori_loop(0, WE // L, vec2, 0)
        pltpu.sync_copy(oe, out_h.at[pl.ds(off, WE)])
        return 0
    lax.fori_loop(0, EPW // WE, win, 0)


def kernel(edge_index, capacity, efeatures, W1, We1, al1, ar1, ae1, b1,
           W2, We2, al2, ar2, ae2, b2):
    src = edge_index[0].astype(_i32)
    dst = edge_index[1].astype(_i32)
    cap = capacity[:, 0]
    efe = efeatures[:N_EDGES, 0]
    efl = jnp.pad(efeatures[N_EDGES:, 0], (0, NP - N_NODES))

    W1h = W1.reshape(2, 16)
    cl1 = jnp.sum(W1h * al1, axis=1)
    cr1 = jnp.sum(W1h * ar1, axis=1)
    ce1 = jnp.sum(We1.reshape(2, 16) * ae1, axis=1)
    bsum = b1[:16] + b1[16:]
    a2l = al2[:, 0]
    a2r = ar2[:, 0]
    ce2 = We2[0] * ae2[:, 0]
    b2s = b2[0] + b2[1]

    def spl2(v):
        return jnp.broadcast_to(v[:, None], (2, 16)).astype(_f32)

    def spl16(v):
        return jnp.broadcast_to(v[:, None], (16, 16)).astype(_f32)

    sc = _k_sumcap(dst, cap)
    v1 = jnp.broadcast_to(sc[None, :], (NC, NP))
    m1 = _k_segmax(src, dst, efe, efl, v1, spl2(cl1), spl2(cr1), spl2(ce1))
    s1 = _k_segsum(src, dst, efe, efl, v1, m1,
                   spl2(cl1), spl2(cr1), spl2(ce1))
    g = _k_node(s1, spl16(W1h[0]), spl16(W1h[1]), spl16(bsum),
                spl16(W2[:, 0]), spl16(W2[:, 1]))
    m2 = _k_segmax(src, dst, efe, efl, g, spl2(a2l), spl2(a2r), spl2(ce2))
    t2 = _k_segsum(src, dst, efe, efl, g, m2, spl2(a2l), spl2(a2r), spl2(ce2))
    out = _k_out(src, dst, t2, jnp.broadcast_to(b2s, (16,)).astype(_f32))
    return out[:, None]
